# baseline (speedup 1.0000x reference)
import jax
import jax.numpy as jnp
from jax import lax
from jax.experimental import pallas as pl
from jax.experimental.pallas import tpu as pltpu

B_, T_ = 4, 576
N = B_ * T_
K = 8192
D = 256
H = D // 2
KB = 512
NBLK = 2
NB = N // NBLK


def _vq_body(x_ref, wt_ref, z_ref, out_ref, ids_ref, c_ref):
    n = x_ref.shape[0]
    x = x_ref[...]
    mu = x[:, :H]
    a = (jnp.sum(mu * mu, axis=1, keepdims=True)
         + jnp.sum(jnp.exp(x[:, H:]), axis=1, keepdims=True))
    best = jnp.full((n, 1), jnp.inf, jnp.float32)
    bidx = jnp.zeros((n, 1), jnp.int32)
    for c in range(K // KB):
        mut = wt_ref[:H, pl.ds(c * KB, KB)]
        w2t = wt_ref[H:, pl.ds(c * KB, KB)]
        invden = 1.0 / (2.0 * jnp.sum(jnp.exp(w2t), axis=0, keepdims=True))
        e = (0.5 * jnp.sum(w2t, axis=0, keepdims=True)
             + jnp.sum(mut * mut, axis=0, keepdims=True) * invden)
        s = lax.dot_general(mu, mut, (((1,), (0,)), ((), ())),
                            preferred_element_type=jnp.float32)
        dist = e + (a - 2.0 * s) * invden
        m = jnp.min(dist, axis=1, keepdims=True)
        kk = lax.broadcasted_iota(jnp.int32, (n, KB), 1) + (c * KB)
        ci = jnp.min(jnp.where(dist <= m, kk, jnp.int32(2**31 - 1)),
                     axis=1, keepdims=True)
        upd = m < best
        bidx = jnp.where(upd, ci, bidx)
        best = jnp.minimum(best, m)
    ids_ref[...] = bidx

    q = jnp.zeros((n, D), jnp.float32)
    for c in range(K // KB):
        w_c = wt_ref[:, pl.ds(c * KB, KB)]
        kk = lax.broadcasted_iota(jnp.int32, (n, KB), 1) + (c * KB)
        oh = (kk == bidx).astype(jnp.float32)
        q = q + lax.dot_general(oh, w_c, (((1,), (1,)), ((), ())),
                                preferred_element_type=jnp.float32)

    d = q - x
    c_ref[...] = jnp.sum(d * d, axis=1, keepdims=True) * (1.0 / D)
    out_ref[...] = q[:, :H] + jnp.exp(0.5 * q[:, H:]) * z_ref[...]


def kernel(input, weight):
    x = input.reshape(N, D)
    wt = weight.T
    z = jax.random.normal(jax.random.fold_in(jax.random.key(0), 123),
                          (B_, T_, H), dtype=jnp.float32).reshape(N, H)
    out, ids2, c = pl.pallas_call(
        _vq_body,
        grid=(NBLK,),
        in_specs=[
            pl.BlockSpec((NB, D), lambda i: (i, 0)),
            pl.BlockSpec((D, K), lambda i: (0, 0)),
            pl.BlockSpec((NB, H), lambda i: (i, 0)),
        ],
        out_specs=(
            pl.BlockSpec((NB, H), lambda i: (i, 0)),
            pl.BlockSpec((NB, 1), lambda i: (i, 0)),
            pl.BlockSpec((NB, 1), lambda i: (i, 0)),
        ),
        out_shape=(jax.ShapeDtypeStruct((N, H), jnp.float32),
                   jax.ShapeDtypeStruct((N, 1), jnp.int32),
                   jax.ShapeDtypeStruct((N, 1), jnp.float32)),
        compiler_params=pltpu.CompilerParams(
            dimension_semantics=("parallel",)),
    )(x, wt, z)
    c = c.reshape(B_, T_)
    return out.reshape(B_, T_, H), ids2.reshape(B_, T_), c, c


# speedup vs baseline: 1.0145x; 1.0145x over previous
import jax
import jax.numpy as jnp
from jax import lax
from jax.experimental import pallas as pl
from jax.experimental.pallas import tpu as pltpu

B_, T_ = 4, 576
N = B_ * T_
K = 8192
D = 256
H = D // 2
KB = 512
NBLK = 2
NB = N // NBLK


def _vq_body(x_ref, wt_ref, z_ref, out_ref, ids_ref, c_ref):
    n = x_ref.shape[0]
    x = x_ref[...]
    mu = x[:, :H]
    a = (jnp.sum(mu * mu, axis=1, keepdims=True)
         + jnp.sum(jnp.exp(x[:, H:]), axis=1, keepdims=True))
    best = jnp.full((n, 1), jnp.inf, jnp.float32)
    bidx = jnp.zeros((n, 1), jnp.int32)
    for c in range(K // KB):
        mut = wt_ref[:H, pl.ds(c * KB, KB)]
        w2t = wt_ref[H:, pl.ds(c * KB, KB)]
        invden = 1.0 / (2.0 * jnp.sum(jnp.exp(w2t), axis=0, keepdims=True))
        e = (0.5 * jnp.sum(w2t, axis=0, keepdims=True)
             + jnp.sum(mut * mut, axis=0, keepdims=True) * invden)
        s = lax.dot_general(mu, mut, (((1,), (0,)), ((), ())),
                            preferred_element_type=jnp.float32)
        dist = e + (a - 2.0 * s) * invden
        m = jnp.min(dist, axis=1, keepdims=True)
        kk = lax.broadcasted_iota(jnp.int32, (n, KB), 1) + (c * KB)
        ci = jnp.min(jnp.where(dist <= m, kk, jnp.int32(2**31 - 1)),
                     axis=1, keepdims=True)
        upd = m < best
        bidx = jnp.where(upd, ci, bidx)
        best = jnp.minimum(best, m)
    ids_ref[...] = bidx

    q = jnp.zeros((n, D), jnp.float32)
    for c in range(K // KB):
        w_c = wt_ref[:, pl.ds(c * KB, KB)]
        kk = lax.broadcasted_iota(jnp.int32, (n, KB), 1) + (c * KB)
        oh = (kk == bidx).astype(jnp.float32)
        q = q + lax.dot_general(oh, w_c, (((1,), (1,)), ((), ())),
                                preferred_element_type=jnp.float32)

    d = q - x
    c_ref[...] = jnp.sum(d * d, axis=1, keepdims=True) * (1.0 / D)
    out_ref[...] = q[:, :H] + jnp.exp(0.5 * q[:, H:]) * z_ref[...]


def kernel(input, weight):
    x = input.reshape(N, D)
    wt = weight.T
    z = jax.random.normal(jax.random.fold_in(jax.random.key(0), 123),
                          (B_, T_, H), dtype=jnp.float32).reshape(N, H)
    out, ids2, c = pl.pallas_call(
        _vq_body,
        out_shape=(jax.ShapeDtypeStruct((N, H), jnp.float32),
                   jax.ShapeDtypeStruct((N, 1), jnp.int32),
                   jax.ShapeDtypeStruct((N, 1), jnp.float32)),
    )(x, wt, z)
    c = c.reshape(B_, T_)
    return out.reshape(B_, T_, H), ids2.reshape(B_, T_), c, c


# speedup vs baseline: 1.2724x; 1.2542x over previous
import jax
import jax.numpy as jnp
from jax import lax
from jax.experimental import pallas as pl

B_, T_ = 4, 576
N = B_ * T_
K = 8192
D = 256
H = D // 2
KB = 512

_Z = jax.random.normal(jax.random.fold_in(jax.random.key(0), 123),
                       (B_, T_, H), dtype=jnp.float32).reshape(N, H)

_NT = (((1,), (1,)), ((), ()))
_NN = (((1,), (0,)), ((), ()))


def _vq_body(x_ref, w_ref, z_ref, out_ref, ids_ref, c_ref):
    n = x_ref.shape[0]
    x = x_ref[...]
    mu = x[:, :H]
    a = (jnp.sum(mu * mu, axis=1, keepdims=True)
         + jnp.sum(jnp.exp(x[:, H:]), axis=1, keepdims=True))
    ones = jnp.ones((1, H), jnp.float32)
    lanes = lax.broadcasted_iota(jnp.int32, (n, KB), 1)
    best = jnp.full((n, 1), jnp.inf, jnp.float32)
    bidx = jnp.zeros((n, 1), jnp.int32)
    for c in range(K // KB):
        w_c = w_ref[pl.ds(c * KB, KB), :]
        wmu = w_c[:, :H]
        w2 = w_c[:, H:]
        sum_exp = lax.dot_general(ones, jnp.exp(w2), _NT,
                                  precision=lax.Precision.HIGHEST,
                                  preferred_element_type=jnp.float32)
        sum_w2 = lax.dot_general(ones, w2, _NT,
                                 precision=lax.Precision.HIGHEST,
                                 preferred_element_type=jnp.float32)
        sum_mu2 = lax.dot_general(ones, wmu * wmu, _NT,
                                  precision=lax.Precision.HIGHEST,
                                  preferred_element_type=jnp.float32)
        invden = 0.5 / sum_exp
        e = 0.5 * sum_w2 + sum_mu2 * invden
        s = lax.dot_general(mu, wmu, _NT,
                            preferred_element_type=jnp.float32)
        dist = (a * invden + e) - s * (2.0 * invden)
        m = jnp.min(dist, axis=1, keepdims=True)
        ci = jnp.min(jnp.where(dist <= m, lanes, jnp.int32(2**31 - 1)),
                     axis=1, keepdims=True) + (c * KB)
        upd = m < best
        bidx = jnp.where(upd, ci, bidx)
        best = jnp.minimum(best, m)
    ids_ref[...] = bidx

    q = jnp.zeros((n, D), jnp.float32)
    for c in range(K // KB):
        w_c = w_ref[pl.ds(c * KB, KB), :]
        oh = (lanes == (bidx - c * KB)).astype(jnp.float32)
        q = q + lax.dot_general(oh, w_c, _NN,
                                preferred_element_type=jnp.float32)

    d = q - x
    c_ref[...] = jnp.sum(d * d, axis=1, keepdims=True) * (1.0 / D)
    out_ref[...] = q[:, :H] + jnp.exp(0.5 * q[:, H:]) * z_ref[...]


def kernel(input, weight):
    x = input.reshape(N, D)
    out, ids2, c = pl.pallas_call(
        _vq_body,
        out_shape=(jax.ShapeDtypeStruct((N, H), jnp.float32),
                   jax.ShapeDtypeStruct((N, 1), jnp.int32),
                   jax.ShapeDtypeStruct((N, 1), jnp.float32)),
    )(x, weight, _Z)
    c = c.reshape(B_, T_)
    return out.reshape(B_, T_, H), ids2.reshape(B_, T_), c, c
